# Initial kernel scaffold; baseline (speedup 1.0000x reference)
import functools

import jax
import jax.numpy as jnp
from jax import lax
from jax.experimental import pallas as pl
from jax.experimental.pallas import tpu as pltpu
from jax.experimental.pallas import tpu_sc as plsc

NUM_EMB = 1000000
EMB_DIM = 32
BATCH = 16384
HIST = 50
B = BATCH * HIST

NC, NS = 2, 16
NW = NC * NS
BPW = B // NW
IDXW = 128
CHUNK = 1280
NSTREAM = CHUNK // IDXW
NSTEP = BPW // CHUNK

PACK = 8
DP = PACK * EMB_DIM
TC_ROWS = 2048


def _sc_gather(table, idx2d):
    mesh = plsc.VectorSubcoreMesh(core_axis_name="c", subcore_axis_name="s")

    @functools.partial(
        pl.kernel,
        out_type=jax.ShapeDtypeStruct((B, EMB_DIM), jnp.float32),
        mesh=mesh,
        scratch_types=[
            pltpu.VMEM((BPW // IDXW, IDXW), jnp.int32),
            pltpu.VMEM((2, CHUNK, EMB_DIM), jnp.float32),
            pltpu.SemaphoreType.DMA,
            pltpu.SemaphoreType.DMA,
        ],
    )
    def k(table_hbm, idx_hbm, out_hbm, idx_v, rows_v, gsem, psem):
        wid = lax.axis_index("s") * NC + lax.axis_index("c")
        rowbase = wid * BPW
        nrows = BPW // IDXW
        pltpu.sync_copy(idx_hbm.at[pl.ds(wid * nrows, nrows)], idx_v)

        def step(s, _):
            for t in range(NSTREAM):
                pltpu.async_copy(
                    table_hbm.at[idx_v.at[s * NSTREAM + t]],
                    rows_v.at[0].at[pl.ds(t * IDXW, IDXW)],
                    gsem,
                )
            for t in range(NSTREAM):
                pltpu.make_async_copy(
                    table_hbm.at[idx_v.at[0]],
                    rows_v.at[0].at[pl.ds(0, IDXW)],
                    gsem,
                ).wait()
            pltpu.async_copy(
                rows_v.at[0],
                out_hbm.at[pl.ds(rowbase + s * CHUNK, CHUNK)],
                psem,
            ).wait()
            return 0

        lax.fori_loop(0, NSTEP, step, 0)

    return k(table, idx2d)


def _mlp_body(x_ref, k1_ref, b1_ref, k2_ref, b2_ref, o_ref):
    h = jnp.dot(x_ref[...], k1_ref[...], preferred_element_type=jnp.float32)
    h = jnp.maximum(h + b1_ref[...], 0.0)
    h = jnp.dot(h, k2_ref[...], preferred_element_type=jnp.float32)
    o_ref[...] = jnp.maximum(h + b2_ref[...], 0.0)


def _tc_mlp(g2, k1, b1t, k2, b2t):
    m = g2.shape[0]
    grid = (m // TC_ROWS,)
    return pl.pallas_call(
        _mlp_body,
        grid=grid,
        in_specs=[
            pl.BlockSpec((TC_ROWS, DP), lambda i: (i, 0)),
            pl.BlockSpec((DP, DP), lambda i: (0, 0)),
            pl.BlockSpec((1, DP), lambda i: (0, 0)),
            pl.BlockSpec((DP, DP), lambda i: (0, 0)),
            pl.BlockSpec((1, DP), lambda i: (0, 0)),
        ],
        out_specs=pl.BlockSpec((TC_ROWS, DP), lambda i: (i, 0)),
        out_shape=jax.ShapeDtypeStruct((m, DP), jnp.float32),
    )(g2, k1, b1t, k2, b2t)


def kernel(x, emb, W1, b1, W2, b2):
    idx2d = x.reshape(B // IDXW, IDXW).astype(jnp.int32)
    g = _sc_gather(emb, idx2d)
    eye = jnp.eye(PACK, dtype=jnp.float32)
    K1 = jnp.kron(eye, W1.T)
    K2 = jnp.kron(eye, W2.T)
    b1t = jnp.tile(b1, PACK).reshape(1, DP)
    b2t = jnp.tile(b2, PACK).reshape(1, DP)
    out = _tc_mlp(g.reshape(B // PACK, DP), K1, b1t, K2, b2t)
    return out.reshape(BATCH, HIST, EMB_DIM)


# speedup vs baseline: 16.9440x; 16.9440x over previous
import functools

import jax
import jax.numpy as jnp
from jax import lax
from jax.experimental import pallas as pl
from jax.experimental.pallas import tpu as pltpu
from jax.experimental.pallas import tpu_sc as plsc

NUM_EMB = 1000000
EMB_DIM = 32
BATCH = 16384
HIST = 50
B = BATCH * HIST

NC, NS = 2, 16
NW = NC * NS
BPW = B // NW
IDXW = 128
CHUNK = 1280
NSTREAM = CHUNK // IDXW
NSTEP = BPW // CHUNK

PACK = 8
DP = PACK * EMB_DIM
TC_ROWS = 2048


def _sc_gather(table, idx2d):
    mesh = plsc.VectorSubcoreMesh(core_axis_name="c", subcore_axis_name="s")

    @functools.partial(
        pl.kernel,
        out_type=jax.ShapeDtypeStruct((B, EMB_DIM), jnp.float32),
        mesh=mesh,
        compiler_params=pltpu.CompilerParams(use_tc_tiling_on_sc=False),
        scratch_types=[
            pltpu.VMEM((BPW // IDXW, IDXW), jnp.int32),
            pltpu.VMEM((2, CHUNK, EMB_DIM), jnp.float32),
            pltpu.SemaphoreType.DMA,
            pltpu.SemaphoreType.DMA,
        ],
    )
    def k(table_hbm, idx_hbm, out_hbm, idx_v, rows_v, gsem, psem):
        wid = lax.axis_index("s") * NC + lax.axis_index("c")
        rowbase = wid * BPW
        nrows = BPW // IDXW
        pltpu.sync_copy(idx_hbm.at[pl.ds(wid * nrows, nrows)], idx_v)

        def step(s, _):
            for t in range(NSTREAM):
                pltpu.async_copy(
                    table_hbm.at[idx_v.at[s * NSTREAM + t]],
                    rows_v.at[0].at[pl.ds(t * IDXW, IDXW)],
                    gsem,
                )
            for t in range(NSTREAM):
                pltpu.make_async_copy(
                    table_hbm.at[idx_v.at[0]],
                    rows_v.at[0].at[pl.ds(0, IDXW)],
                    gsem,
                ).wait()
            pltpu.async_copy(
                rows_v.at[0],
                out_hbm.at[pl.ds(rowbase + s * CHUNK, CHUNK)],
                psem,
            ).wait()
            return 0

        lax.fori_loop(0, NSTEP, step, 0)

    return k(table, idx2d)


def _mlp_body(x_ref, k1_ref, b1_ref, k2_ref, b2_ref, o_ref):
    h = jnp.dot(x_ref[...], k1_ref[...], preferred_element_type=jnp.float32)
    h = jnp.maximum(h + b1_ref[...], 0.0)
    h = jnp.dot(h, k2_ref[...], preferred_element_type=jnp.float32)
    o_ref[...] = jnp.maximum(h + b2_ref[...], 0.0)


def _tc_mlp(g2, k1, b1t, k2, b2t):
    m = g2.shape[0]
    grid = (m // TC_ROWS,)
    return pl.pallas_call(
        _mlp_body,
        grid=grid,
        in_specs=[
            pl.BlockSpec((TC_ROWS, DP), lambda i: (i, 0)),
            pl.BlockSpec((DP, DP), lambda i: (0, 0)),
            pl.BlockSpec((1, DP), lambda i: (0, 0)),
            pl.BlockSpec((DP, DP), lambda i: (0, 0)),
            pl.BlockSpec((1, DP), lambda i: (0, 0)),
        ],
        out_specs=pl.BlockSpec((TC_ROWS, DP), lambda i: (i, 0)),
        out_shape=jax.ShapeDtypeStruct((m, DP), jnp.float32),
    )(g2, k1, b1t, k2, b2t)


def kernel(x, emb, W1, b1, W2, b2):
    idx2d = x.reshape(B // IDXW, IDXW).astype(jnp.int32)
    g = _sc_gather(emb, idx2d)
    eye = jnp.eye(PACK, dtype=jnp.float32)
    K1 = jnp.kron(eye, W1.T)
    K2 = jnp.kron(eye, W2.T)
    b1t = jnp.tile(b1, PACK).reshape(1, DP)
    b2t = jnp.tile(b2, PACK).reshape(1, DP)
    out = _tc_mlp(g.reshape(B // PACK, DP), K1, b1t, K2, b2t)
    return out.reshape(BATCH, HIST, EMB_DIM)
